# baseline (speedup 1.0000x reference)
import functools
import math

import jax
import jax.numpy as jnp
import numpy as np
from jax.experimental import pallas as pl
from jax.experimental.pallas import tpu as pltpu

_ROT_A = (13, 15, 26, 6)
_ROT_B = (17, 29, 16, 24)
_TINY = np.float32(np.finfo(np.float32).tiny)


def _np_threefry2x32(k0, k1, x0, x1):
    x0 = np.asarray(x0, np.uint32).copy()
    x1 = np.asarray(x1, np.uint32).copy()
    ks = [np.uint32(k0), np.uint32(k1),
          np.uint32(np.uint32(k0) ^ np.uint32(k1) ^ np.uint32(0x1BD11BDA))]
    rots = [_ROT_A, _ROT_B]
    x0 = (x0 + ks[0]).astype(np.uint32)
    x1 = (x1 + ks[1]).astype(np.uint32)
    for i in range(5):
        for r in rots[i % 2]:
            x0 = (x0 + x1).astype(np.uint32)
            x1 = ((x1 << np.uint32(r)) | (x1 >> np.uint32(32 - r))).astype(np.uint32)
            x1 = (x1 ^ x0).astype(np.uint32)
        x0 = (x0 + ks[(i + 1) % 3]).astype(np.uint32)
        x1 = (x1 + ks[(i + 2) % 3] + np.uint32(i + 1)).astype(np.uint32)
    return x0, x1


def _field_keys(n_fields):
    idx = np.arange(n_fields, dtype=np.uint32)
    o0, o1 = _np_threefry2x32(0, 42, np.zeros(n_fields, np.uint32), idx)
    return o0, o1


def _tf_rounds(x0, x1, k0, k1):
    k2 = k0 ^ k1 ^ jnp.uint32(0x1BD11BDA)
    ks = (k0, k1, k2)
    x0 = x0 + ks[0]
    x1 = x1 + ks[1]
    for i in range(5):
        rots = _ROT_A if i % 2 == 0 else _ROT_B
        for r in rots:
            x0 = x0 + x1
            x1 = (x1 << jnp.uint32(r)) | (x1 >> jnp.uint32(32 - r))
            x1 = x1 ^ x0
        x0 = x0 + ks[(i + 1) % 3]
        x1 = x1 + ks[(i + 2) % 3] + jnp.uint32(i + 1)
    return x0, x1


def _sample_kernel(logits_ref, k0_ref, k1_ref, act_ref, lp_ref,
                   bv_s, bi_s, bl_s, m_s, s_s, *, block, vocab, nb):
    b = pl.program_id(1)
    rows = logits_ref.shape[0]
    base = b * block
    lane = jax.lax.broadcasted_iota(jnp.int32, (rows, block), 1) + base
    valid = lane < vocab

    l = logits_ref[...]
    neg_inf = jnp.float32(-jnp.inf)
    lm = jnp.where(valid, l, neg_inf)

    x0 = jnp.zeros((rows, block), jnp.uint32)
    x1 = lane.astype(jnp.uint32)
    o0, o1 = _tf_rounds(x0, x1, k0_ref[...], k1_ref[...])
    bits = o0 ^ o1

    uf = pltpu.bitcast((bits >> jnp.uint32(9)) | jnp.uint32(0x3F800000),
                       jnp.float32) - jnp.float32(1.0)
    u = jnp.maximum(_TINY, uf * jnp.float32(1.0 - float(_TINY)) + _TINY)
    g = -jnp.log(-jnp.log(u))

    v = jnp.where(valid, l + g, neg_inf)

    bv = jnp.max(v, axis=1, keepdims=True)
    big = jnp.int32(2**31 - 1)
    bi = jnp.min(jnp.where(v == bv, lane, big), axis=1, keepdims=True)
    bl = jnp.max(jnp.where(lane == bi, lm, neg_inf), axis=1, keepdims=True)

    blm = jnp.max(lm, axis=1, keepdims=True)
    bs = jnp.sum(jnp.where(valid, jnp.exp(lm - blm), 0.0), axis=1,
                 keepdims=True)

    @pl.when(b == 0)
    def _init():
        bv_s[...] = bv
        bi_s[...] = bi
        bl_s[...] = bl
        m_s[...] = blm
        s_s[...] = bs

    @pl.when(b > 0)
    def _merge():
        pv, pi, plg = bv_s[...], bi_s[...], bl_s[...]
        take = (bv > pv) | ((bv == pv) & (bi < pi))
        bv_s[...] = jnp.where(take, bv, pv)
        bi_s[...] = jnp.where(take, bi, pi)
        bl_s[...] = jnp.where(take, bl, plg)
        pm, ps = m_s[...], s_s[...]
        mn = jnp.maximum(pm, blm)
        m_s[...] = mn
        s_s[...] = ps * jnp.exp(pm - mn) + bs * jnp.exp(blm - mn)

    @pl.when(b == nb - 1)
    def _emit():
        act_ref[...] = bi_s[...]
        lp_ref[...] = bl_s[...] - (m_s[...] + jnp.log(s_s[...]))


@functools.partial(jax.jit, static_argnums=())
def kernel(logits):
    n_fields, vocab = logits.shape
    rows = 8
    block = 4096
    nfb = math.ceil(n_fields / rows)
    nb = math.ceil(vocab / block)
    nfp = nfb * rows

    k0np, k1np = _field_keys(n_fields)
    k0 = jnp.asarray(np.pad(k0np, (0, nfp - n_fields)).reshape(nfp, 1))
    k1 = jnp.asarray(np.pad(k1np, (0, nfp - n_fields)).reshape(nfp, 1))

    act, lp = pl.pallas_call(
        functools.partial(_sample_kernel, block=block, vocab=vocab, nb=nb),
        grid=(nfb, nb),
        in_specs=[
            pl.BlockSpec((rows, block), lambda f, b: (f, b)),
            pl.BlockSpec((rows, 1), lambda f, b: (f, 0)),
            pl.BlockSpec((rows, 1), lambda f, b: (f, 0)),
        ],
        out_specs=[
            pl.BlockSpec((rows, 1), lambda f, b: (f, 0)),
            pl.BlockSpec((rows, 1), lambda f, b: (f, 0)),
        ],
        out_shape=[
            jax.ShapeDtypeStruct((nfp, 1), jnp.int32),
            jax.ShapeDtypeStruct((nfp, 1), jnp.float32),
        ],
        scratch_shapes=[
            pltpu.VMEM((rows, 1), jnp.float32),
            pltpu.VMEM((rows, 1), jnp.int32),
            pltpu.VMEM((rows, 1), jnp.float32),
            pltpu.VMEM((rows, 1), jnp.float32),
            pltpu.VMEM((rows, 1), jnp.float32),
        ],
    )(logits, k0, k1)

    action = act[:n_fields, 0]
    log_prob = lp[:n_fields, 0].sum()
    return (action, log_prob, jnp.float32(1.0))
